# Initial kernel scaffold; baseline (speedup 1.0000x reference)
import jax
import jax.numpy as jnp
from jax.experimental import pallas as pl


def kernel(data, edge_index, W1, b1, g1, be1, W2, b2, g2, be2, W3, b3):
    raise NotImplementedError("write your pallas kernel here")


# speedup vs baseline: 17.5808x; 17.5808x over previous
import functools

import jax
import jax.numpy as jnp
from jax import lax
from jax.experimental import pallas as pl
from jax.experimental.pallas import tpu as pltpu
from jax.experimental.pallas import tpu_sc as plsc

N = 10000
E = 320000
D_IN = 128
H = 64
C = 40

NC = 2
NS = 16
NW = NC * NS
CHUNK = 128
CH = -(-E // (NW * CHUNK))
E_PAD = NW * CH * CHUNK
NP = 10112
RPS = NP // NS

_mesh = plsc.VectorSubcoreMesh(core_axis_name="c", subcore_axis_name="s")


@functools.partial(
    pl.kernel,
    out_type=jax.ShapeDtypeStruct((NC, NP, 16), jnp.float32),
    mesh=_mesh,
    compiler_params=pltpu.CompilerParams(use_tc_tiling_on_sc=False),
    scratch_types=[
        pltpu.VMEM((CH, CHUNK), jnp.int32),
        pltpu.VMEM((CHUNK, 16), jnp.float32),
        pltpu.VMEM_SHARED((NP, 16), jnp.float32),
        pltpu.SemaphoreType.DMA,
    ],
)
def _deg_sc(col_hbm, ones_hbm, zeros_hbm, out_hbm, col_v, ones_v, deg_sh, sem):
    c = lax.axis_index("c")
    s = lax.axis_index("s")
    pltpu.sync_copy(col_hbm.at[c, s], col_v)
    pltpu.sync_copy(ones_hbm, ones_v)
    pltpu.sync_copy(zeros_hbm.at[pl.ds(s * RPS, RPS)],
                    deg_sh.at[pl.ds(s * RPS, RPS)])
    plsc.subcore_barrier()

    def body(j, carry):
        pltpu.sync_copy(ones_v, deg_sh.at[col_v.at[j]], add=True)
        return carry

    lax.fori_loop(0, CH, body, 0)
    plsc.subcore_barrier()
    pltpu.sync_copy(deg_sh.at[pl.ds(s * RPS, RPS)],
                    out_hbm.at[c, pl.ds(s * RPS, RPS)])


@functools.partial(
    pl.kernel,
    out_type=jax.ShapeDtypeStruct((NC, NP, H), jnp.float32),
    mesh=_mesh,
    compiler_params=pltpu.CompilerParams(use_tc_tiling_on_sc=False),
    scratch_types=[
        pltpu.VMEM((CH, CHUNK), jnp.int32),
        pltpu.VMEM((CH, CHUNK), jnp.int32),
        pltpu.VMEM((CHUNK, H), jnp.float32),
        pltpu.VMEM_SHARED((NP, H), jnp.float32),
        pltpu.SemaphoreType.DMA,
    ],
)
def _agg_sc(s_hbm, row_hbm, col_hbm, zeros_hbm, out_hbm,
            row_v, col_v, buf, agg_sh, sem):
    c = lax.axis_index("c")
    s = lax.axis_index("s")
    pltpu.sync_copy(row_hbm.at[c, s], row_v)
    pltpu.sync_copy(col_hbm.at[c, s], col_v)
    pltpu.sync_copy(zeros_hbm.at[pl.ds(s * RPS, RPS)],
                    agg_sh.at[pl.ds(s * RPS, RPS)])
    plsc.subcore_barrier()

    def body(j, carry):
        pltpu.async_copy(s_hbm.at[row_v.at[j]], buf, sem).wait()
        pltpu.sync_copy(buf, agg_sh.at[col_v.at[j]], add=True)
        return carry

    lax.fori_loop(0, CH, body, 0)
    plsc.subcore_barrier()
    pltpu.sync_copy(agg_sh.at[pl.ds(s * RPS, RPS)],
                    out_hbm.at[c, pl.ds(s * RPS, RPS)])


def _stage1_body(deg_ref, data_ref, w1_ref, s_ref, dinv_ref):
    cnt = deg_ref[0, 0:N, 0:1] + deg_ref[1, 0:N, 0:1]
    dinv = lax.rsqrt(cnt + 1.0)
    h = jnp.dot(data_ref[...], w1_ref[...], preferred_element_type=jnp.float32)
    s_ref[...] = h * dinv
    dinv_ref[...] = dinv


_stage1 = pl.pallas_call(
    _stage1_body,
    out_shape=(jax.ShapeDtypeStruct((N, H), jnp.float32),
               jax.ShapeDtypeStruct((N, 1), jnp.float32)),
)


def _stage2_body(p_ref, s_ref, dinv_ref, b_ref, g_ref, be_ref, w2_ref, out_ref):
    dinv = dinv_ref[...]
    agg = p_ref[0, 0:N, :] + p_ref[1, 0:N, :] + s_ref[...]
    x = jnp.maximum(agg * dinv + b_ref[...], 0.0)
    m = jnp.mean(x, axis=0, keepdims=True)
    v = jnp.mean((x - m) ** 2, axis=0, keepdims=True)
    x = (x - m) * lax.rsqrt(v + 1e-5) * g_ref[...] + be_ref[...]
    x = jnp.maximum(x, 0.0)
    h = jnp.dot(x, w2_ref[...], preferred_element_type=jnp.float32)
    out_ref[...] = h * dinv


_stage2 = pl.pallas_call(
    _stage2_body,
    out_shape=jax.ShapeDtypeStruct((N, H), jnp.float32),
)


def _stage3_body(p_ref, s_ref, dinv_ref, b_ref, g_ref, be_ref, w3_ref, b3_ref,
                 out_ref):
    dinv = dinv_ref[...]
    agg = p_ref[0, 0:N, :] + p_ref[1, 0:N, :] + s_ref[...]
    x = jnp.maximum(agg * dinv + b_ref[...], 0.0)
    m = jnp.mean(x, axis=0, keepdims=True)
    v = jnp.mean((x - m) ** 2, axis=0, keepdims=True)
    x = (x - m) * lax.rsqrt(v + 1e-5) * g_ref[...] + be_ref[...]
    h = jnp.dot(x, w3_ref[...], preferred_element_type=jnp.float32)
    out_ref[...] = jnp.maximum(h + b3_ref[...], 0.0)


_stage3 = pl.pallas_call(
    _stage3_body,
    out_shape=jax.ShapeDtypeStruct((N, C), jnp.float32),
)


def kernel(data, edge_index, W1, b1, g1, be1, W2, b2, g2, be2, W3, b3):
    pad = E_PAD - E
    rowp = jnp.concatenate(
        [edge_index[0], jnp.zeros((pad,), jnp.int32)]).reshape(NC, NS, CH, CHUNK)
    colp = jnp.concatenate(
        [edge_index[1], jnp.full((pad,), N, jnp.int32)]).reshape(NC, NS, CH, CHUNK)
    ones16 = jnp.zeros((CHUNK, 16), jnp.float32).at[:, 0].set(1.0)
    zeros16 = jnp.zeros((NP, 16), jnp.float32)
    zeros64 = jnp.zeros((NP, H), jnp.float32)

    deg = _deg_sc(colp, ones16, zeros16)
    s1, dinv = _stage1(deg, data, W1)
    p1 = _agg_sc(s1, rowp, colp, zeros64)
    s2 = _stage2(p1, s1, dinv, b1.reshape(1, H), g1.reshape(1, H),
                 be1.reshape(1, H), W2)
    p2 = _agg_sc(s2, rowp, colp, zeros64)
    out = _stage3(p2, s2, dinv, b2.reshape(1, H), g2.reshape(1, H),
                  be2.reshape(1, H), W3, b3.reshape(1, C))
    return out
